# baseline (speedup 1.0000x reference)
import dataclasses
import functools

import jax
import jax.numpy as jnp
from jax import lax
from jax.experimental import pallas as pl
from jax.experimental.pallas import tpu as pltpu
from jax.experimental.pallas import tpu_sc as plsc

L = 16
NC = 2
NS = 16
D = 256
DH = D // 2
V = 512
VR = V + 1
MASK_ID = V * V
N_TOK = 4 * 32 * 1024
TPW = N_TOK // NS
C = 128
NCHUNK = TPW // C
HI = -65536


def _pack_half(tab, h):
    half = tab[:, h * DH:(h + 1) * DH].astype(jnp.bfloat16)
    r = half.reshape(VR, DH // 32, 2, L)
    bits = lax.bitcast_convert_type(r, jnp.uint16).astype(jnp.uint32)
    words = bits[:, :, 0, :] | (bits[:, :, 1, :] << 16)
    return lax.bitcast_convert_type(words, jnp.int32).reshape(VR * (DH // 2))


def _make_sc_embed():
    mesh = plsc.VectorSubcoreMesh(core_axis_name="c", subcore_axis_name="s")
    cp = pltpu.CompilerParams()
    if "needs_layout_passes" in pltpu.CompilerParams.__dataclass_fields__:
        cp = dataclasses.replace(cp, needs_layout_passes=False)

    @functools.partial(
        pl.kernel,
        out_type=jax.ShapeDtypeStruct((N_TOK, D), jnp.float32),
        mesh=mesh,
        compiler_params=cp,
        scratch_types=[
            pltpu.VMEM((VR * (DH // 2),), jnp.int32),
            pltpu.VMEM((VR * (DH // 2),), jnp.int32),
            pltpu.VMEM((C, DH), jnp.float32),
            pltpu.VMEM((C, DH), jnp.float32),
            pltpu.VMEM((TPW,), jnp.int32),
            pltpu.VMEM((TPW,), jnp.int32),
            pltpu.SMEM((C,), jnp.int32),
            pltpu.SemaphoreType.DMA,
            pltpu.SemaphoreType.DMA,
            pltpu.SemaphoreType.DMA,
        ],
    )
    def sc_embed(ids_hbm, t0_hbm, t1_hbm, out_hbm,
                 t0_v, t1_v, oa, ob, ids_v, idxc_v,
                 soff, st, swa, swb):
        h = lax.axis_index("c")
        span = lax.axis_index("s")
        base = span * TPW

        cp0 = pltpu.make_async_copy(t0_hbm.at[h], t0_v, st)
        cp1 = pltpu.make_async_copy(t1_hbm.at[h], t1_v, st)
        cp0.start()
        cp1.start()
        pltpu.sync_copy(ids_hbm.at[pl.ds(base, TPW)], ids_v)

        @plsc.parallel_loop(0, TPW // L)
        def _(g):
            sl = pl.ds(g * L, L)
            ids16 = ids_v[sl]
            is_m = ids16 == MASK_ID
            r0 = jnp.where(is_m, V, lax.bitwise_and(ids16, V - 1))
            r1 = jnp.where(
                is_m, V,
                lax.bitwise_and(lax.shift_right_logical(ids16, 9), V - 1))
            idxc_v[sl] = lax.bitwise_or(lax.shift_left(r0, 6),
                                        lax.shift_left(r1, 22))

        sets = ((oa, swa), (ob, swb))

        def out_desc(i, st_):
            o, sw = st_
            return pltpu.make_async_copy(
                o, out_hbm.at[pl.ds(base + i * C, C),
                              pl.ds(h * DH, DH)], sw)

        cp0.wait()
        cp1.wait()

        @pl.loop(0, NCHUNK, step=2)
        def _(j):
            for b in range(2):
                i = j + b
                o, sw = sets[b]

                @pl.when(j > 0)
                def _():
                    out_desc(i - 2, sets[b]).wait()

                @plsc.parallel_loop(0, C, step=L)
                def _(tt):
                    offs = idxc_v[pl.ds(i * C + tt, L)]
                    for t in range(L):
                        soff[tt + t] = offs[t]

                for g in range(DH // 32):
                    @plsc.parallel_loop(0, C)
                    def _(t, g=g):
                        oc = soff[t]
                        o0 = lax.bitwise_and(oc, 0xFFFF)
                        o1 = lax.shift_right_logical(oc, 16)
                        w0 = t0_v[pl.ds(o0 + g * L, L)]
                        w1 = t1_v[pl.ds(o1 + g * L, L)]
                        s = (plsc.bitcast(w0, jnp.bfloat16) +
                             plsc.bitcast(w1, jnp.bfloat16))
                        lo, hi = plsc.unpack(
                            s, format=plsc.PackFormat.INTERLEAVED)
                        o[t, pl.ds(g * 32, L)] = lo
                        o[t, pl.ds(g * 32 + L, L)] = hi

                out_desc(i, sets[b]).start()

        for b in range(2):
            out_desc(NCHUNK - 2 + b, sets[b]).wait()

    return sc_embed


_SC_EMBED = _make_sc_embed()


def kernel(input_ids, embed0, embed1, mask_token_embed):
    ids = input_ids.reshape(N_TOK)
    t0 = jnp.concatenate([embed0, mask_token_embed], axis=0)
    t1 = jnp.concatenate([embed1, jnp.zeros((1, D), jnp.float32)], axis=0)
    t0p = jnp.stack([_pack_half(t0, 0), _pack_half(t0, 1)])
    t1p = jnp.stack([_pack_half(t1, 0), _pack_half(t1, 1)])
    out = _SC_EMBED(ids, t0p, t1p)
    return out.reshape(*input_ids.shape, D)


# speedup vs baseline: 1.2800x; 1.2800x over previous
import dataclasses
import functools

import jax
import jax.numpy as jnp
from jax import lax
from jax.experimental import pallas as pl
from jax.experimental.pallas import tpu as pltpu
from jax.experimental.pallas import tpu_sc as plsc

L = 16
NC = 2
NS = 16
D = 256
DH = D // 2
V = 512
VR = V + 1
MASK_ID = V * V
N_TOK = 4 * 32 * 1024
TPW = N_TOK // NS
C = 128
NCHUNK = TPW // C
HI = -65536


def _pack_half(tab, h):
    half = tab[:, h * DH:(h + 1) * DH].astype(jnp.bfloat16)
    r = half.reshape(VR, DH // 32, 2, L)
    bits = lax.bitcast_convert_type(r, jnp.uint16).astype(jnp.uint32)
    words = bits[:, :, 0, :] | (bits[:, :, 1, :] << 16)
    return lax.bitcast_convert_type(words, jnp.int32).reshape(VR * (DH // 2))


def _make_sc_embed():
    mesh = plsc.VectorSubcoreMesh(core_axis_name="c", subcore_axis_name="s")
    cp = pltpu.CompilerParams()
    if "needs_layout_passes" in pltpu.CompilerParams.__dataclass_fields__:
        cp = dataclasses.replace(cp, needs_layout_passes=False)

    @functools.partial(
        pl.kernel,
        out_type=jax.ShapeDtypeStruct((N_TOK, D), jnp.float32),
        mesh=mesh,
        compiler_params=cp,
        scratch_types=[
            pltpu.VMEM((VR * (DH // 2),), jnp.int32),
            pltpu.VMEM((VR * (DH // 2),), jnp.int32),
            pltpu.VMEM((C, DH), jnp.float32),
            pltpu.VMEM((C, DH), jnp.float32),
            pltpu.VMEM((TPW,), jnp.int32),
            pltpu.VMEM((TPW,), jnp.int32),
            pltpu.SMEM((C,), jnp.int32),
            pltpu.SemaphoreType.DMA,
            pltpu.SemaphoreType.DMA,
            pltpu.SemaphoreType.DMA,
        ],
    )
    def sc_embed(ids_hbm, t0_hbm, t1_hbm, out_hbm,
                 t0_v, t1_v, oa, ob, ids_v, idxc_v,
                 soff, st, swa, swb):
        h = lax.axis_index("c")
        span = lax.axis_index("s")
        base = span * TPW

        cp0 = pltpu.make_async_copy(t0_hbm.at[h], t0_v, st)
        cp1 = pltpu.make_async_copy(t1_hbm.at[h], t1_v, st)
        cp0.start()
        cp1.start()
        pltpu.sync_copy(ids_hbm.at[pl.ds(base, TPW)], ids_v)

        @plsc.parallel_loop(0, TPW // L)
        def _(g):
            sl = pl.ds(g * L, L)
            ids16 = ids_v[sl]
            is_m = ids16 == MASK_ID
            r0 = jnp.where(is_m, V, lax.bitwise_and(ids16, V - 1))
            r1 = jnp.where(
                is_m, V,
                lax.bitwise_and(lax.shift_right_logical(ids16, 9), V - 1))
            idxc_v[sl] = lax.bitwise_or(lax.shift_left(r0, 6),
                                        lax.shift_left(r1, 22))

        sets = ((oa, swa), (ob, swb))

        def out_desc(i, st_):
            o, sw = st_
            return pltpu.make_async_copy(
                o, out_hbm.at[pl.ds(base + i * C, C),
                              pl.ds(h * DH, DH)], sw)

        cp0.wait()
        cp1.wait()

        @pl.loop(0, NCHUNK, step=2)
        def _(j):
            for b in range(2):
                i = j + b
                o, sw = sets[b]

                @pl.when(j > 0)
                def _():
                    out_desc(i - 2, sets[b]).wait()

                @plsc.parallel_loop(0, C, step=L)
                def _(tt):
                    offs = idxc_v[pl.ds(i * C + tt, L)]
                    for t in range(L):
                        soff[tt + t] = offs[t]

                @plsc.parallel_loop(0, C, step=2)
                def _(t):
                    oc0 = soff[t]
                    oc1 = soff[t + 1]
                    oab = (lax.bitwise_and(oc0, 0xFFFF),
                           lax.shift_right_logical(oc0, 16),
                           lax.bitwise_and(oc1, 0xFFFF),
                           lax.shift_right_logical(oc1, 16))
                    for g in range(DH // 32):
                        for u in range(2):
                            w0 = t0_v[pl.ds(oab[2 * u] + g * L, L)]
                            w1 = t1_v[pl.ds(oab[2 * u + 1] + g * L, L)]
                            s = (plsc.bitcast(w0, jnp.bfloat16) +
                                 plsc.bitcast(w1, jnp.bfloat16))
                            lo, hi = plsc.unpack(
                                s, format=plsc.PackFormat.INTERLEAVED)
                            o[t + u, pl.ds(g * 32, L)] = lo
                            o[t + u, pl.ds(g * 32 + L, L)] = hi

                out_desc(i, sets[b]).start()

        for b in range(2):
            out_desc(NCHUNK - 2 + b, sets[b]).wait()

    return sc_embed


_SC_EMBED = _make_sc_embed()


def kernel(input_ids, embed0, embed1, mask_token_embed):
    ids = input_ids.reshape(N_TOK)
    t0 = jnp.concatenate([embed0, mask_token_embed], axis=0)
    t1 = jnp.concatenate([embed1, jnp.zeros((1, D), jnp.float32)], axis=0)
    t0p = jnp.stack([_pack_half(t0, 0), _pack_half(t0, 1)])
    t1p = jnp.stack([_pack_half(t1, 0), _pack_half(t1, 1)])
    out = _SC_EMBED(ids, t0p, t1p)
    return out.reshape(*input_ids.shape, D)


# speedup vs baseline: 1.9179x; 1.4984x over previous
import dataclasses
import functools

import jax
import jax.numpy as jnp
from jax import lax
from jax.experimental import pallas as pl
from jax.experimental.pallas import tpu as pltpu
from jax.experimental.pallas import tpu_sc as plsc

L = 16
NC = 2
NS = 16
D = 256
DH = D // 2
V = 512
VR = V + 1
MASK_ID = V * V
N_TOK = 4 * 32 * 1024
TPW = N_TOK // NS
C = 128
NCHUNK = TPW // C


def _pack_half(tab, h):
    half = tab[:, h * DH:(h + 1) * DH].astype(jnp.bfloat16)
    r = half.reshape(VR, DH // 32, 2, L)
    bits = lax.bitcast_convert_type(r, jnp.uint16).astype(jnp.uint32)
    words = bits[:, :, 0, :] | (bits[:, :, 1, :] << 16)
    return lax.bitcast_convert_type(words, jnp.int32).reshape(VR * (DH // 2))


def _make_sc_embed():
    mesh = plsc.VectorSubcoreMesh(core_axis_name="c", subcore_axis_name="s")
    cp = pltpu.CompilerParams()
    if "needs_layout_passes" in pltpu.CompilerParams.__dataclass_fields__:
        cp = dataclasses.replace(cp, needs_layout_passes=False)

    @functools.partial(
        pl.kernel,
        out_type=jax.ShapeDtypeStruct((N_TOK, D), jnp.float32),
        mesh=mesh,
        compiler_params=cp,
        scratch_types=[
            pltpu.VMEM((VR * (DH // 2),), jnp.int32),
            pltpu.VMEM((VR * (DH // 2),), jnp.int32),
            pltpu.VMEM((C, DH), jnp.float32),
            pltpu.VMEM((C, DH), jnp.float32),
            pltpu.VMEM((TPW,), jnp.int32),
            pltpu.VMEM((TPW,), jnp.int32),
            pltpu.SMEM((C,), jnp.int32),
            pltpu.SemaphoreType.DMA,
            pltpu.SemaphoreType.DMA,
            pltpu.SemaphoreType.DMA,
        ],
    )
    def sc_embed(ids_hbm, t0_hbm, t1_hbm, out_hbm,
                 t0_v, t1_v, oa, ob, ids_v, idxc_v,
                 soff, st, swa, swb):
        h = lax.axis_index("c")
        span = lax.axis_index("s")
        base = span * TPW

        cp0 = pltpu.make_async_copy(t0_hbm.at[h], t0_v, st)
        cp1 = pltpu.make_async_copy(t1_hbm.at[h], t1_v, st)
        cp0.start()
        cp1.start()
        pltpu.sync_copy(ids_hbm.at[pl.ds(base, TPW)], ids_v)

        @plsc.parallel_loop(0, TPW // L)
        def _(g):
            sl = pl.ds(g * L, L)
            ids16 = ids_v[sl]
            is_m = ids16 == MASK_ID
            r0 = jnp.where(is_m, V, lax.bitwise_and(ids16, V - 1))
            r1 = jnp.where(
                is_m, V,
                lax.bitwise_and(lax.shift_right_logical(ids16, 9), V - 1))
            idxc_v[sl] = lax.bitwise_or(lax.shift_left(r0, 6),
                                        lax.shift_left(r1, 22))

        sets = ((oa, swa), (ob, swb))

        def out_desc(i, st_):
            o, sw = st_
            return pltpu.make_async_copy(
                o, out_hbm.at[pl.ds(base + i * C, C),
                              pl.ds(h * DH, DH)], sw)

        cp0.wait()
        cp1.wait()

        @pl.loop(0, NCHUNK, step=2)
        def _(j):
            for b in range(2):
                i = j + b
                o, sw = sets[b]

                @pl.when(j > 0)
                def _():
                    out_desc(i - 2, sets[b]).wait()

                @plsc.parallel_loop(0, C, step=L)
                def _(tt):
                    offs = idxc_v[pl.ds(i * C + tt, L)]
                    for t in range(L):
                        soff[tt + t] = offs[t]

                @plsc.parallel_loop(0, C)
                def _(t):
                    oc = soff[t]
                    o0 = lax.bitwise_and(oc, 0xFFFF)
                    o1 = lax.shift_right_logical(oc, 16)
                    for g in range(DH // 32):
                        w0 = t0_v[pl.ds(o0 + g * L, L)]
                        w1 = t1_v[pl.ds(o1 + g * L, L)]
                        s = (plsc.bitcast(w0, jnp.bfloat16) +
                             plsc.bitcast(w1, jnp.bfloat16))
                        lo, hi = plsc.unpack(
                            s, format=plsc.PackFormat.INTERLEAVED)
                        o[t, pl.ds(g * 32, L)] = lo
                        o[t, pl.ds(g * 32 + L, L)] = hi

                out_desc(i, sets[b]).start()

        for b in range(2):
            out_desc(NCHUNK - 2 + b, sets[b]).wait()

    return sc_embed


_SC_EMBED = _make_sc_embed()


def kernel(input_ids, embed0, embed1, mask_token_embed):
    ids = input_ids.reshape(N_TOK)
    t0 = jnp.concatenate([embed0, mask_token_embed], axis=0)
    t1 = jnp.concatenate([embed1, jnp.zeros((1, D), jnp.float32)], axis=0)
    t0p = jnp.stack([_pack_half(t0, 0), _pack_half(t0, 1)])
    t1p = jnp.stack([_pack_half(t1, 0), _pack_half(t1, 1)])
    out = _SC_EMBED(ids, t0p, t1p)
    return out.reshape(*input_ids.shape, D)
